# baseline (speedup 1.0000x reference)
import jax
import jax.numpy as jnp
from jax import lax
from jax.experimental import pallas as pl
from jax.experimental.pallas import tpu as pltpu

T = 128
E = 64
D = 768
F = 3072
RT = 8
NRT = T // RT

_INV_SQRT2 = 0.7071067811865476
_DN0 = (((0,), (0,)), ((), ()))


def _routing(xx, wr, br):
    logits = jnp.dot(xx, wr, preferred_element_type=jnp.float32)
    logits = logits + br
    m = jnp.max(logits, axis=1, keepdims=True)
    p = jnp.exp(logits - m)
    p = p / jnp.sum(p, axis=1, keepdims=True)

    cols = lax.broadcasted_iota(jnp.int32, (T, E), 1)
    m1 = jnp.max(p, axis=1, keepdims=True)
    i1 = jnp.min(jnp.where(p == m1, cols, E), axis=1, keepdims=True)
    pm = jnp.where(cols == i1, -1.0, p)
    m2 = jnp.max(pm, axis=1, keepdims=True)
    i2 = jnp.min(jnp.where(pm == m2, cols, E), axis=1, keepdims=True)
    s = m1 + m2
    w1 = m1 / s
    w2 = m2 / s

    oh1 = (cols == i1).astype(jnp.float32)
    oh2 = (cols == i2).astype(jnp.float32)

    rows_t = lax.broadcasted_iota(jnp.int32, (T, T), 0)
    cols_t = lax.broadcasted_iota(jnp.int32, (T, T), 1)
    ltri = (rows_t > cols_t).astype(jnp.float32)
    p1 = jnp.dot(ltri, oh1, preferred_element_type=jnp.float32)
    p2 = jnp.dot(ltri, oh2, preferred_element_type=jnp.float32)
    c1 = jnp.sum(oh1, axis=0, keepdims=True)

    rank1 = jnp.sum(p1 * oh1, axis=1, keepdims=True)
    rank2 = jnp.sum((p2 + c1) * oh2, axis=1, keepdims=True)

    def enc(q):
        fl = jnp.floor(q * 0.125)
        return (q - 8.0 * fl) * 16.0 + fl

    slots = lax.broadcasted_iota(jnp.int32, (T, T), 1).astype(jnp.float32)
    s1 = (enc(rank1) == slots).astype(jnp.float32)
    s2 = (enc(rank2) == slots).astype(jnp.float32)
    tok = lax.broadcasted_iota(jnp.int32, (T, 1), 0).astype(jnp.float32)

    tab = lax.dot_general(oh1, s1 * tok, _DN0, preferred_element_type=jnp.float32)
    tab = tab + lax.dot_general(oh2, s2 * tok, _DN0, preferred_element_type=jnp.float32)
    ptab = lax.dot_general(oh1, s1 * w1, _DN0, preferred_element_type=jnp.float32)
    ptab = ptab + lax.dot_general(oh2, s2 * w2, _DN0, preferred_element_type=jnp.float32)

    cnt = c1 + jnp.sum(oh2, axis=0, keepdims=True)

    ones_t = jnp.ones((T, 1), jnp.float32)
    cnt_col = lax.dot_general(oh1 + oh2, ones_t, _DN0,
                              preferred_element_type=jnp.float32)
    e_col = lax.broadcasted_iota(jnp.int32, (E, 1), 0).astype(jnp.float32)
    hit_col = jnp.where((cnt_col > 0.0) | (e_col == 0.0), 1.0, 0.0)
    er = lax.broadcasted_iota(jnp.int32, (E, E), 0)
    ec = lax.broadcasted_iota(jnp.int32, (E, E), 1)
    ltriE = (ec < er).astype(jnp.float32)
    pos_col = jnp.dot(ltriE, hit_col, preferred_element_type=jnp.float32)
    p_iotaE = lax.broadcasted_iota(jnp.int32, (E, E), 1).astype(jnp.float32)
    mm = jnp.where(pos_col == p_iotaE, hit_col, 0.0)
    order = lax.dot_general(e_col, mm, _DN0,
                            preferred_element_type=jnp.float32)
    nhit = jnp.sum(hit_col, axis=0, keepdims=True)
    return tab, ptab, cnt, order, nhit


def _body(x_ref, wr_ref, br_ref, b1_ref, b2_ref, w1_hbm, w2_hbm, out_ref,
          tab_s, ptab_s, w1buf, w2buf, ord_s, cnt_s, nh_s, sem1, sem2):
    i = pl.program_id(0)

    @pl.when(i == 0)
    def _init():
        pltpu.make_async_copy(w1_hbm.at[0], w1buf.at[0], sem1.at[0]).start()
        pltpu.make_async_copy(w2_hbm.at[0], w2buf.at[0], sem2.at[0]).start()
        out_ref[...] = jnp.zeros_like(out_ref)

        tab, ptab, cnt, order, nhit = _routing(
            x_ref[...], wr_ref[...], br_ref[...])
        tab_s[...] = tab.reshape(E, RT, NRT)
        ptab_s[...] = ptab.reshape(E, RT, NRT)
        for e in range(E):
            ord_s[e] = order[0, e].astype(jnp.int32)
            cnt_s[e] = cnt[0, e].astype(jnp.int32)
        nh_s[0] = nhit[0, 0].astype(jnp.int32)

    nh = nh_s[0]
    b = lax.rem(i, 2)
    nb = lax.rem(i + 1, 2)

    @pl.when(i + 1 < nh)
    def _issue_next():
        en = ord_s[i + 1]
        pltpu.make_async_copy(w1_hbm.at[en], w1buf.at[nb], sem1.at[nb]).start()
        pltpu.make_async_copy(w2_hbm.at[en], w2buf.at[nb], sem2.at[nb]).start()

    @pl.when(i < nh)
    def _compute():
        eo = ord_s[i]
        pltpu.make_async_copy(w1_hbm.at[eo], w1buf.at[b], sem1.at[b]).wait()
        pltpu.make_async_copy(w2_hbm.at[eo], w2buf.at[b], sem2.at[b]).wait()

        cnt = cnt_s[eo]
        xv = x_ref[...].astype(jnp.bfloat16)
        onehot_e = (lax.broadcasted_iota(jnp.int32, (1, E), 1) == eo).astype(
            jnp.float32)
        b1c = jnp.dot(onehot_e, b1_ref[...], preferred_element_type=jnp.float32)
        b2r = jnp.dot(onehot_e, b2_ref[...], preferred_element_type=jnp.float32)
        w1 = w1buf[b].astype(jnp.bfloat16)
        w2 = w2buf[b].astype(jnp.bfloat16)
        ito = lax.broadcasted_iota(jnp.int32, (RT, T), 1).astype(jnp.float32)

        for r in range(NRT):
            @pl.when(cnt > r * RT)
            def _(r=r):
                col = tab_s[eo, :, r:r + 1]
                pcol = ptab_s[eo, :, r:r + 1]
                g = (col == ito).astype(jnp.bfloat16)
                xg = jnp.dot(g, xv, preferred_element_type=jnp.float32)
                h = jnp.dot(xg.astype(jnp.bfloat16), w1,
                            preferred_element_type=jnp.float32) + b1c
                h = 0.5 * h * (1.0 + lax.erf(h * _INV_SQRT2))
                part = jnp.dot(h.astype(jnp.bfloat16), w2,
                               preferred_element_type=jnp.float32)
                part = part + b2r
                contrib = (pcol * part).astype(jnp.bfloat16)
                out_ref[...] += lax.dot_general(
                    g, contrib, _DN0, preferred_element_type=jnp.float32)


@jax.jit
def kernel(x, Wr, br, W1, b1, W2, b2):
    B, S, _ = x.shape
    x2 = x.reshape(T, D)

    out = pl.pallas_call(
        _body,
        grid=(E,),
        in_specs=[
            pl.BlockSpec((T, D), lambda i: (0, 0)),
            pl.BlockSpec((D, E), lambda i: (0, 0)),
            pl.BlockSpec((1, E), lambda i: (0, 0)),
            pl.BlockSpec((E, F), lambda i: (0, 0)),
            pl.BlockSpec((E, D), lambda i: (0, 0)),
            pl.BlockSpec(memory_space=pl.ANY),
            pl.BlockSpec(memory_space=pl.ANY),
        ],
        out_specs=pl.BlockSpec((T, D), lambda i: (0, 0)),
        out_shape=jax.ShapeDtypeStruct((T, D), jnp.float32),
        scratch_shapes=[
            pltpu.VMEM((E, RT, NRT), jnp.float32),
            pltpu.VMEM((E, RT, NRT), jnp.float32),
            pltpu.VMEM((2, D, F), jnp.float32),
            pltpu.VMEM((2, F, D), jnp.float32),
            pltpu.SMEM((E,), jnp.int32),
            pltpu.SMEM((E,), jnp.int32),
            pltpu.SMEM((1,), jnp.int32),
            pltpu.SemaphoreType.DMA((2,)),
            pltpu.SemaphoreType.DMA((2,)),
        ],
    )(x2, Wr, br.reshape(1, E), b1, b2, W1, W2)

    return out.reshape(B, S, D)


# speedup vs baseline: 1.0224x; 1.0224x over previous
import jax
import jax.numpy as jnp
from jax import lax
from jax.experimental import pallas as pl
from jax.experimental.pallas import tpu as pltpu

T = 128
E = 64
D = 768
F = 3072

_INV_SQRT2 = 0.7071067811865476
_DN0 = (((0,), (0,)), ((), ()))


def _routing(xx, wr, br):
    logits = jnp.dot(xx, wr, preferred_element_type=jnp.float32)
    logits = logits + br
    m = jnp.max(logits, axis=1, keepdims=True)
    p = jnp.exp(logits - m)
    p = p / jnp.sum(p, axis=1, keepdims=True)

    cols = lax.broadcasted_iota(jnp.int32, (T, E), 1)
    m1 = jnp.max(p, axis=1, keepdims=True)
    i1 = jnp.min(jnp.where(p == m1, cols, E), axis=1, keepdims=True)
    pm = jnp.where(cols == i1, -1.0, p)
    m2 = jnp.max(pm, axis=1, keepdims=True)
    i2 = jnp.min(jnp.where(pm == m2, cols, E), axis=1, keepdims=True)
    s = m1 + m2
    w1 = m1 / s
    w2 = m2 / s

    oh1 = (cols == i1).astype(jnp.float32)
    oh2 = (cols == i2).astype(jnp.float32)

    rows_t = lax.broadcasted_iota(jnp.int32, (T, T), 0)
    cols_t = lax.broadcasted_iota(jnp.int32, (T, T), 1)
    ltri = (rows_t > cols_t).astype(jnp.float32)
    p1 = jnp.dot(ltri, oh1, preferred_element_type=jnp.float32)
    p2 = jnp.dot(ltri, oh2, preferred_element_type=jnp.float32)
    c1 = jnp.sum(oh1, axis=0, keepdims=True)

    rank1 = jnp.sum(p1 * oh1, axis=1, keepdims=True)
    rank2 = jnp.sum((p2 + c1) * oh2, axis=1, keepdims=True)

    slots = lax.broadcasted_iota(jnp.int32, (T, T), 1).astype(jnp.float32)
    s1 = (rank1 == slots).astype(jnp.float32)
    s2 = (rank2 == slots).astype(jnp.float32)
    tok = lax.broadcasted_iota(jnp.int32, (T, 1), 0).astype(jnp.float32)

    tab = lax.dot_general(oh1, s1 * tok, _DN0, preferred_element_type=jnp.float32)
    tab = tab + lax.dot_general(oh2, s2 * tok, _DN0, preferred_element_type=jnp.float32)
    ptab = lax.dot_general(oh1, s1 * w1, _DN0, preferred_element_type=jnp.float32)
    ptab = ptab + lax.dot_general(oh2, s2 * w2, _DN0, preferred_element_type=jnp.float32)

    cnt = c1 + jnp.sum(oh2, axis=0, keepdims=True)

    ones_t = jnp.ones((T, 1), jnp.float32)
    cnt_col = lax.dot_general(oh1 + oh2, ones_t, _DN0,
                              preferred_element_type=jnp.float32)
    e_col = lax.broadcasted_iota(jnp.int32, (E, 1), 0).astype(jnp.float32)
    hit_col = jnp.where((cnt_col > 0.0) | (e_col == 0.0), 1.0, 0.0)
    er = lax.broadcasted_iota(jnp.int32, (E, E), 0)
    ec = lax.broadcasted_iota(jnp.int32, (E, E), 1)
    ltriE = (ec < er).astype(jnp.float32)
    pos_col = jnp.dot(ltriE, hit_col, preferred_element_type=jnp.float32)
    p_iotaE = lax.broadcasted_iota(jnp.int32, (E, E), 1).astype(jnp.float32)
    mm = jnp.where(pos_col == p_iotaE, hit_col, 0.0)
    order = lax.dot_general(e_col, mm, _DN0,
                            preferred_element_type=jnp.float32)
    nhit = jnp.sum(hit_col, axis=0, keepdims=True)
    return tab, ptab, cnt, order, nhit


def _body(x_ref, wr_ref, br_ref, b1_ref, b2_ref, w1_hbm, w2_hbm, out_ref,
          tab_s, ptab_s, xbf_s, w1buf, w2buf, ord_s, cnt_s, nh_s, sem1, sem2):
    i = pl.program_id(0)

    @pl.when(i == 0)
    def _init():
        pltpu.make_async_copy(w1_hbm.at[0], w1buf.at[0], sem1.at[0]).start()
        pltpu.make_async_copy(w2_hbm.at[0], w2buf.at[0], sem2.at[0]).start()
        out_ref[...] = jnp.zeros_like(out_ref)

        tab, ptab, cnt, order, nhit = _routing(
            x_ref[...], wr_ref[...], br_ref[...])
        tab_s[...] = tab
        ptab_s[...] = ptab
        xbf_s[...] = x_ref[...].astype(jnp.bfloat16)
        for e in range(E):
            ord_s[e] = order[0, e].astype(jnp.int32)
            cnt_s[e] = cnt[0, e].astype(jnp.int32)
        nh_s[0] = nhit[0, 0].astype(jnp.int32)

    nh = nh_s[0]
    b = lax.rem(i, 2)
    nb = lax.rem(i + 1, 2)

    @pl.when(i + 1 < nh)
    def _issue_next():
        en = ord_s[i + 1]
        pltpu.make_async_copy(w1_hbm.at[en], w1buf.at[nb], sem1.at[nb]).start()
        pltpu.make_async_copy(w2_hbm.at[en], w2buf.at[nb], sem2.at[nb]).start()

    @pl.when(i < nh)
    def _compute():
        eo = ord_s[i]
        pltpu.make_async_copy(w1_hbm.at[eo], w1buf.at[b], sem1.at[b]).wait()
        pltpu.make_async_copy(w2_hbm.at[eo], w2buf.at[b], sem2.at[b]).wait()

        cnt = cnt_s[eo]
        b1c = b1_ref[pl.dslice(eo, 1), :]
        b2r = b2_ref[pl.dslice(eo, 1), :]
        tokcol = lax.broadcasted_iota(jnp.int32, (T, 1), 0).astype(jnp.float32)

        for RS in (8, 16, 32, 64, 128):
            lo = RS // 2 if RS > 8 else 0

            @pl.when((cnt > lo) & (cnt <= RS))
            def _(RS=RS):
                ids = tab_s[pl.dslice(eo, 1), 0:RS]
                pw = ptab_s[pl.dslice(eo, 1), 0:RS]
                gT = (tokcol == ids).astype(jnp.float32)
                xg = lax.dot_general(gT.astype(jnp.bfloat16), xbf_s[...],
                                     _DN0, preferred_element_type=jnp.float32)
                h = jnp.dot(xg.astype(jnp.bfloat16),
                            w1buf[b].astype(jnp.bfloat16),
                            preferred_element_type=jnp.float32) + b1c
                h = 0.5 * h * (1.0 + lax.erf(h * _INV_SQRT2))
                part = jnp.dot(h.astype(jnp.bfloat16),
                               w2buf[b].astype(jnp.bfloat16),
                               preferred_element_type=jnp.float32)
                part = part + b2r
                gw = (gT * pw).astype(jnp.bfloat16)
                out_ref[...] += jnp.dot(gw, part.astype(jnp.bfloat16),
                                        preferred_element_type=jnp.float32)


@jax.jit
def kernel(x, Wr, br, W1, b1, W2, b2):
    B, S, _ = x.shape
    x2 = x.reshape(T, D)

    out = pl.pallas_call(
        _body,
        grid=(E,),
        in_specs=[
            pl.BlockSpec((T, D), lambda i: (0, 0)),
            pl.BlockSpec((D, E), lambda i: (0, 0)),
            pl.BlockSpec((1, E), lambda i: (0, 0)),
            pl.BlockSpec((E, F), lambda i: (0, 0)),
            pl.BlockSpec((E, D), lambda i: (0, 0)),
            pl.BlockSpec(memory_space=pl.ANY),
            pl.BlockSpec(memory_space=pl.ANY),
        ],
        out_specs=pl.BlockSpec((T, D), lambda i: (0, 0)),
        out_shape=jax.ShapeDtypeStruct((T, D), jnp.float32),
        scratch_shapes=[
            pltpu.VMEM((E, T), jnp.float32),
            pltpu.VMEM((E, T), jnp.float32),
            pltpu.VMEM((T, D), jnp.bfloat16),
            pltpu.VMEM((2, D, F), jnp.float32),
            pltpu.VMEM((2, F, D), jnp.float32),
            pltpu.SMEM((E,), jnp.int32),
            pltpu.SMEM((E,), jnp.int32),
            pltpu.SMEM((1,), jnp.int32),
            pltpu.SemaphoreType.DMA((2,)),
            pltpu.SemaphoreType.DMA((2,)),
        ],
    )(x2, Wr, br.reshape(1, E), b1, b2, W1, W2)

    return out.reshape(B, S, D)
